# Initial kernel scaffold; baseline (speedup 1.0000x reference)
import jax
import jax.numpy as jnp
from jax.experimental import pallas as pl


def kernel(featidx, featval, hist_reserve_prices, weights_linear, weights_factorized, fm_intercept, dist_shape):
    raise NotImplementedError("write your pallas kernel here")


# speedup vs baseline: 2.3554x; 2.3554x over previous
import functools

import jax
import jax.numpy as jnp
from jax import lax
from jax.experimental import pallas as pl
from jax.experimental.pallas import tpu as pltpu
from jax.experimental.pallas import tpu_sc as plsc

B, F, V, K = 16384, 26, 1000000, 32
NC, NS = 2, 16
NW = NC * NS
SPW = B // NW
CS = 64
NCH = SPW // CS
IPC = CS * F
NOPS = IPC // 128
ROWS2D = (B * F) // 128


def _sc_body(idx_hbm, val_hbm, wl_hbm, wf_hbm, raw_hbm,
             idx_v, val_v, lin_v, rows_v, raw_v, mat_v, sem):
    wid = lax.axis_index("s") * NC + lax.axis_index("c")
    row0 = wid * (SPW * F // 128)
    flat0 = wid * (SPW * F)

    def chunk_body(ch, carry):
        pltpu.sync_copy(idx_hbm.at[pl.ds(flat0 + ch * IPC, IPC)],
                        idx_v.at[pl.ds(0, IPC)])
        pltpu.sync_copy(val_hbm.at[pl.ds(flat0 + ch * IPC, IPC)],
                        val_v.at[pl.ds(0, IPC)])
        for j in range(NOPS):
            pltpu.make_async_copy(wf_hbm.at[idx_v.at[pl.ds(j * 128, 128)]],
                                  rows_v.at[pl.ds(j * 128, 128)], sem).start()
            pltpu.make_async_copy(wl_hbm.at[idx_v.at[pl.ds(j * 128, 128)]],
                                  lin_v.at[pl.ds(j * 128, 128)], sem).start()
        for j in range(NOPS):
            pltpu.make_async_copy(wf_hbm.at[idx_v.at[pl.ds(j * 128, 128)]],
                                  rows_v.at[pl.ds(j * 128, 128)], sem).wait()
            pltpu.make_async_copy(wl_hbm.at[idx_v.at[pl.ds(j * 128, 128)]],
                                  lin_v.at[pl.ds(j * 128, 128)], sem).wait()

        lanes = lax.iota(jnp.int32, 16)

        def lane_body(g, l, carry3):
            cf = (g * 16 + l) * F
            vv0 = val_v[pl.ds(cf, 16)]
            vv1 = val_v[pl.ds(cf + 16, 16)]
            acc0 = jnp.zeros((16,), jnp.float32)
            acc1 = jnp.zeros((16,), jnp.float32)
            aux0 = jnp.zeros((16,), jnp.float32)
            aux1 = jnp.zeros((16,), jnp.float32)
            for f in range(F):
                v = vv0[f] if f < 16 else vv1[f - 16]
                e0 = rows_v[cf + f, pl.ds(0, 16)] * v
                e1 = rows_v[cf + f, pl.ds(16, 16)] * v
                acc0 = acc0 + e0
                acc1 = acc1 + e1
                aux0 = aux0 + e0 * e0
                aux1 = aux1 + e1 * e1
            lv0 = lin_v[pl.ds(cf, 16)] * vv0
            lv1 = lin_v[pl.ds(cf + 16, 16)] * vv1
            lv1 = jnp.where(lanes < F - 16, lv1, 0.0)
            comb = lv0 + lv1 + 0.5 * (acc0 * acc0 + acc1 * acc1
                                      - (aux0 + aux1))
            mat_v[pl.ds(l * 16, 16)] = comb
            return carry3

        def group_body(g, carry2):
            lax.fori_loop(0, 16, functools.partial(lane_body, g), 0)
            acc = jnp.zeros((16,), jnp.float32)
            for l in range(16):
                acc = acc + plsc.load_gather(mat_v, [lanes * 16 + l])
            raw_v[pl.ds(g * 16, 16)] = acc
            return carry2

        lax.fori_loop(0, CS // 16, group_body, 0)
        pltpu.sync_copy(raw_v, raw_hbm.at[pl.ds(wid * SPW + ch * CS, CS)])
        return carry

    lax.fori_loop(0, NCH, chunk_body, 0)


def _sc_compute_raw(idx2d, val2d, wl, wf):
    mesh = plsc.VectorSubcoreMesh(core_axis_name="c", subcore_axis_name="s")
    f = functools.partial(
        pl.kernel,
        out_type=jax.ShapeDtypeStruct((B,), jnp.float32),
        mesh=mesh,
        compiler_params=pltpu.CompilerParams(needs_layout_passes=False,
                                             use_tc_tiling_on_sc=False),
        scratch_types=[
            pltpu.VMEM((IPC + 16,), jnp.int32),
            pltpu.VMEM((IPC + 16,), jnp.float32),
            pltpu.VMEM((IPC + 16,), jnp.float32),
            pltpu.VMEM((IPC, K), jnp.float32),
            pltpu.VMEM((CS,), jnp.float32),
            pltpu.VMEM((256,), jnp.float32),
            pltpu.SemaphoreType.DMA,
        ],
    )(_sc_body)
    return f(idx2d, val2d, wl, wf)


def _tc_tail(int_ref, shape_ref, raw_ref, hist_ref, p_ref, bin_ref):
    x = raw_ref[...] + int_ref[0, 0]
    scales = jax.nn.softplus(x)
    t = hist_ref[...]
    p = 1.0 - jnp.exp(-jnp.power(t / scales, shape_ref[0, 0]))
    p_ref[...] = p
    bin_ref[...] = jnp.where(p >= 0.5, 1.0, 0.0)


def kernel(featidx, featval, hist_reserve_prices, weights_linear,
           weights_factorized, fm_intercept, dist_shape):
    idxflat = featidx.astype(jnp.int32).reshape(B * F)
    valflat = featval.reshape(B * F)
    raw = _sc_compute_raw(idxflat, valflat, weights_linear, weights_factorized)

    raw2 = raw.reshape(128, 128)
    hist2 = hist_reserve_prices.reshape(128, 128)
    i2 = fm_intercept.reshape(1, 1)
    d2 = dist_shape.reshape(1, 1)
    p2, b2 = pl.pallas_call(
        _tc_tail,
        in_specs=[
            pl.BlockSpec(memory_space=pltpu.SMEM),
            pl.BlockSpec(memory_space=pltpu.SMEM),
            pl.BlockSpec(memory_space=pltpu.VMEM),
            pl.BlockSpec(memory_space=pltpu.VMEM),
        ],
        out_specs=[
            pl.BlockSpec(memory_space=pltpu.VMEM),
            pl.BlockSpec(memory_space=pltpu.VMEM),
        ],
        out_shape=[
            jax.ShapeDtypeStruct((128, 128), jnp.float32),
            jax.ShapeDtypeStruct((128, 128), jnp.float32),
        ],
    )(i2, d2, raw2, hist2)
    return (p2.reshape(B), b2.reshape(B))
